# baseline (speedup 1.0000x reference)
import functools

import jax
import jax.numpy as jnp
from jax import lax
from jax.experimental import pallas as pl
from jax.experimental.pallas import tpu as pltpu
from jax.experimental.pallas import tpu_sc as plsc

_NUM_CORES = 2
_NUM_SUBCORES = 16
_LANES = 16
_CHUNK = 128
_ROWS_PER_SLICE = 4


def kernel(token_ids, tok_table, pos_table):
    B, S = token_ids.shape
    V, E = tok_table.shape
    n_chunks = S // _CHUNK
    quarter = S // 4
    width = E * _ROWS_PER_SLICE

    table2 = tok_table.reshape(V // _ROWS_PER_SLICE, width)
    qs = (token_ids // _ROWS_PER_SLICE).reshape(B, n_chunks, _CHUNK)
    rs = (token_ids % _ROWS_PER_SLICE) * E
    pt = pos_table.T

    mesh = plsc.VectorSubcoreMesh(
        core_axis_name="c",
        subcore_axis_name="s",
        num_cores=_NUM_CORES,
        num_subcores=_NUM_SUBCORES,
    )

    @functools.partial(
        pl.kernel,
        out_type=jax.ShapeDtypeStruct((B, E, S), jnp.float32),
        mesh=mesh,
        scratch_types=[
            pltpu.VMEM((n_chunks, _CHUNK), jnp.int32),
            pltpu.VMEM((S,), jnp.int32),
            pltpu.VMEM((E, S), jnp.float32),
            pltpu.VMEM((2, _CHUNK, width), jnp.float32),
            pltpu.VMEM((E, quarter), jnp.float32),
            pltpu.SemaphoreType.DMA,
        ],
        compiler_params=pltpu.CompilerParams(
            use_tc_tiling_on_sc=True, needs_layout_passes=False
        ),
    )
    def run(qs_hbm, rs_hbm, t2_hbm, pt_hbm, out_hbm,
            idx_v, off_v, buf, stage, pos_v, sem):
        w = lax.axis_index("s") * _NUM_CORES + lax.axis_index("c")

        pltpu.sync_copy(qs_hbm.at[w], idx_v)
        pltpu.sync_copy(rs_hbm.at[w], off_v)

        pltpu.async_copy(t2_hbm.at[idx_v.at[0]], stage.at[0], sem)
        pltpu.async_copy(t2_hbm.at[idx_v.at[1]], stage.at[1], sem)

        lanes = lax.iota(jnp.int32, _LANES)

        def chunk_body(c, carry):
            par = lax.rem(c, 2)

            @pl.when(lax.rem(c, 4) == 0)
            def _():
                pltpu.sync_copy(
                    pt_hbm.at[:, pl.ds((c // 4) * quarter, quarter)], pos_v
                )

            pltpu.make_async_copy(
                t2_hbm.at[pl.ds(0, _CHUNK)], stage.at[0], sem
            ).wait()

            par_vec = lanes * 0 + par

            def extract(k, kc):
                base = c * _CHUNK + k * _LANES
                rows = k * _LANES + lanes
                cols = off_v[pl.ds(base, _LANES)]
                pbase = lax.rem(c, 4) * _CHUNK + k * _LANES
                for e in range(E):
                    vals = plsc.load_gather(stage, [par_vec, rows, cols + e])
                    pv = pos_v[e, pl.ds(pbase, _LANES)]
                    buf[e, pl.ds(base, _LANES)] = vals + pv
                return kc

            lax.fori_loop(0, _CHUNK // _LANES, extract, 0)

            @pl.when(c < n_chunks - 2)
            def _():
                pltpu.async_copy(
                    t2_hbm.at[idx_v.at[c + 2]], stage.at[par], sem
                )

            return carry

        lax.fori_loop(0, n_chunks, chunk_body, 0)

        pltpu.sync_copy(buf, out_hbm.at[w])

    out3 = run(qs, rs, table2, pt)
    return jnp.transpose(out3, (0, 2, 1))


# speedup vs baseline: 1.0178x; 1.0178x over previous
import functools

import jax
import jax.numpy as jnp
from jax import lax
from jax.experimental import pallas as pl
from jax.experimental.pallas import tpu as pltpu
from jax.experimental.pallas import tpu_sc as plsc

_NUM_CORES = 2
_NUM_SUBCORES = 16
_LANES = 16
_CHUNK = 128


def kernel(token_ids, tok_table, pos_table):
    B, S = token_ids.shape
    V, E = tok_table.shape
    n_chunks = S // _CHUNK
    half = S // 2

    ids3 = token_ids.reshape(B, n_chunks, _CHUNK)

    mesh = plsc.VectorSubcoreMesh(
        core_axis_name="c",
        subcore_axis_name="s",
        num_cores=_NUM_CORES,
        num_subcores=_NUM_SUBCORES,
    )

    @functools.partial(
        pl.kernel,
        out_type=jax.ShapeDtypeStruct((B, S, E), jnp.float32),
        mesh=mesh,
        scratch_types=[
            pltpu.VMEM((n_chunks, _CHUNK), jnp.int32),
            pltpu.VMEM((S, E), jnp.float32),
            pltpu.VMEM((half, E), jnp.float32),
            pltpu.SemaphoreType.DMA,
        ],
        compiler_params=pltpu.CompilerParams(use_tc_tiling_on_sc=False),
    )
    def run(ids_hbm, tok_hbm, pos_hbm, out_hbm, idx_v, buf, pos_v, sem):
        w = lax.axis_index("s") * _NUM_CORES + lax.axis_index("c")

        pltpu.sync_copy(ids_hbm.at[w], idx_v)

        copies = []
        for c in range(n_chunks):
            copies.append(
                pltpu.async_copy(
                    tok_hbm.at[idx_v.at[c]],
                    buf.at[pl.ds(c * _CHUNK, _CHUNK)],
                    sem,
                )
            )
        for cp in copies:
            cp.wait()

        for h in range(2):
            pltpu.sync_copy(pos_hbm.at[pl.ds(h * half, half)], pos_v)

            def body(r, carry, h=h):
                row = h * half + r
                for q in range(E // _LANES):
                    sl = pl.ds(q * _LANES, _LANES)
                    buf[row, sl] = buf[row, sl] + pos_v[r, sl]
                return carry

            lax.fori_loop(0, half, body, 0, unroll=4)

        pltpu.sync_copy(buf, out_hbm.at[w])

    return run(ids3, tok_table, pos_table)
